# baseline (speedup 1.0000x reference)
import jax
import jax.numpy as jnp
from jax import lax
from jax.experimental import pallas as pl
from jax.experimental.pallas import tpu as pltpu
from jax.experimental.pallas import tpu_sc as plsc

B, P, D = 4, 8192, 768
N_ROWS = B * P
NC, NS, L = 2, 16, 16
NW = NC * NS

SC_ROWS = 16384
TC_ROWS = N_ROWS - SC_ROWS

ROWS_PER_W = SC_ROWS // NW
C = 16
NCHUNK = ROWS_PER_W // C
NSLOT = 4
VPR = D // L

R = 256
NBLK = TC_ROWS // R
TC_BLK0 = SC_ROWS // R
WAVE = 64


def _sc_body(x_hbm, idx_hbm, pe_hbm, out_hbm, idx_v, xs, pes,
             sems_in, sems_out):
    wid = lax.axis_index("s") * NC + lax.axis_index("c")
    base0 = wid * ROWS_PER_W
    pltpu.sync_copy(idx_hbm.at[pl.ds(base0, ROWS_PER_W)], idx_v)

    def issue(j, b):
        pltpu.async_copy(
            pe_hbm.at[idx_v.at[pl.ds(j * C, C)]], pes[b], sems_in[b]
        )
        pltpu.async_copy(
            x_hbm.at[pl.ds(base0 + j * C, C)], xs[b], sems_in[b]
        )

    def drain_in(b):
        pltpu.make_async_copy(x_hbm.at[pl.ds(0, C)], xs[b], sems_in[b]).wait()
        pltpu.make_async_copy(x_hbm.at[pl.ds(0, C)], pes[b], sems_in[b]).wait()

    def drain_out(b):
        pltpu.make_async_copy(xs[b], out_hbm.at[pl.ds(0, C)], sems_out[b]).wait()

    def add_chunk(b):
        x_v, pe_v = xs[b], pes[b]

        @plsc.parallel_loop(0, C, step=1, unroll=2)
        def _row(r):
            for c in range(VPR):
                sl = pl.ds(c * L, L)
                x_v[r, sl] = x_v[r, sl] + pe_v[r, sl]

    def store(j, b):
        pltpu.async_copy(xs[b], out_hbm.at[pl.ds(base0 + j * C, C)], sems_out[b])

    for s in range(NSLOT - 1):
        issue(s, s)

    def body(k, carry):
        for s in range(NSLOT):
            j = NSLOT * k + s
            t = (s + NSLOT - 1) % NSLOT

            @pl.when((j >= 1) & (j < NCHUNK - NSLOT + 1))
            def _():
                drain_out(t)

            @pl.when(j < NCHUNK - NSLOT + 1)
            def _():
                issue(j + NSLOT - 1, t)

            drain_in(s)
            add_chunk(s)
            store(j, s)
        return carry

    lax.fori_loop(0, NCHUNK // NSLOT, body, 0)
    for s in range(NSLOT):
        drain_out(s)


def _sc_call(x2d, idx1d, pe2d):
    mesh = plsc.VectorSubcoreMesh(
        core_axis_name="c", subcore_axis_name="s", num_cores=NC, num_subcores=NS
    )

    def entry(x_hbm, idx_hbm, pe_hbm, out_hbm, idx_v,
              x0, x1, x2, x3, pe0, pe1, pe2, pe3,
              si0, si1, si2, si3, so0, so1, so2, so3):
        _sc_body(x_hbm, idx_hbm, pe_hbm, out_hbm, idx_v,
                 (x0, x1, x2, x3), (pe0, pe1, pe2, pe3),
                 (si0, si1, si2, si3), (so0, so1, so2, so3))

    f = pl.kernel(
        entry,
        out_type=jax.ShapeDtypeStruct((SC_ROWS, D), jnp.float32),
        mesh=mesh,
        scratch_types=[pltpu.VMEM((ROWS_PER_W,), jnp.int32)]
        + [pltpu.VMEM((C, D), jnp.float32)] * (2 * NSLOT)
        + [pltpu.SemaphoreType.DMA] * (2 * NSLOT),
    )
    return f(x2d, idx1d, pe2d)


def _tc_body(idx_s, x_ref, pe_hbm, out_ref, pe_buf, sem):
    blk = pl.program_id(0)
    base = blk * R
    descs = [None] * R
    for w in range(R // WAVE):
        for r in range(WAVE):
            row = w * WAVE + r
            g = idx_s[base + row]
            descs[row] = pltpu.make_async_copy(
                pe_hbm.at[pl.ds(g, 1)], pe_buf.at[pl.ds(row, 1)], sem
            )
            descs[row].start()
        if w > 0:
            for r in range(WAVE):
                descs[(w - 1) * WAVE + r].wait()
    for r in range(WAVE):
        descs[R - WAVE + r].wait()
    out_ref[...] = x_ref[...] + pe_buf[...]


def _tc_call(x2d, idx_tc, pe2d):
    grid_spec = pltpu.PrefetchScalarGridSpec(
        num_scalar_prefetch=1,
        grid=(NBLK,),
        in_specs=[
            pl.BlockSpec((R, D), lambda i, idx: (TC_BLK0 + i, 0)),
            pl.BlockSpec(memory_space=pltpu.MemorySpace.HBM),
        ],
        out_specs=pl.BlockSpec((R, D), lambda i, idx: (i, 0)),
        scratch_shapes=[
            pltpu.VMEM((R, D), jnp.float32),
            pltpu.SemaphoreType.DMA,
        ],
    )
    return pl.pallas_call(
        _tc_body,
        grid_spec=grid_spec,
        out_shape=jax.ShapeDtypeStruct((TC_ROWS, D), jnp.float32),
    )(idx_tc, x2d, pe2d)


@jax.jit
def _pe_add(x2d, idx1d, pe2d):
    out_sc = _sc_call(x2d, idx1d, pe2d)
    out_tc = _tc_call(x2d, idx1d[SC_ROWS:], pe2d)
    return jnp.concatenate([out_sc, out_tc], axis=0)


def kernel(x, indices, pe):
    out = _pe_add(
        x.reshape(N_ROWS, D), indices.reshape(N_ROWS), pe.reshape(P, D)
    )
    return out.reshape(B, P, D)


# speedup vs baseline: 2.3300x; 2.3300x over previous
import jax
import jax.numpy as jnp
from jax import lax
from jax.experimental import pallas as pl
from jax.experimental.pallas import tpu as pltpu
from jax.experimental.pallas import tpu_sc as plsc

B, P, D = 4, 8192, 768
N_ROWS = B * P
NC, NS, L = 2, 16, 16
NW = NC * NS
ROWS_PER_W = N_ROWS // NW
C = 16
NCHUNK = ROWS_PER_W // C
NSLOT = 4
LA = 2
VPR = D // L


def _sc_body(x_hbm, idx_hbm, pe_hbm, out_hbm, idx_v, xs, pes,
             sems_in, sems_out):
    wid = lax.axis_index("s") * NC + lax.axis_index("c")
    base0 = wid * ROWS_PER_W
    pltpu.sync_copy(idx_hbm.at[pl.ds(base0, ROWS_PER_W)], idx_v)

    def issue(j, b):
        pltpu.async_copy(
            pe_hbm.at[idx_v.at[pl.ds(j * C, C)]], pes[b], sems_in[b]
        )
        pltpu.async_copy(
            x_hbm.at[pl.ds(base0 + j * C, C)], xs[b], sems_in[b]
        )

    def drain_in(b):
        pltpu.make_async_copy(x_hbm.at[pl.ds(0, C)], xs[b], sems_in[b]).wait()
        pltpu.make_async_copy(x_hbm.at[pl.ds(0, C)], pes[b], sems_in[b]).wait()

    def drain_out(b):
        pltpu.make_async_copy(pes[b], out_hbm.at[pl.ds(0, C)], sems_out[b]).wait()

    def add_chunk(b):
        x_v, pe_v = xs[b], pes[b]

        @plsc.parallel_loop(0, C, step=1, unroll=2)
        def _row(r):
            for c in range(VPR):
                sl = pl.ds(c * L, L)
                pe_v[r, sl] = x_v[r, sl] + pe_v[r, sl]

    def store(j, b):
        pltpu.async_copy(pes[b], out_hbm.at[pl.ds(base0 + j * C, C)], sems_out[b])

    for s in range(LA):
        issue(s, s)

    def body(k, carry):
        for s in range(NSLOT):
            j = NSLOT * k + s
            t = (s + LA) % NSLOT

            @pl.when((j >= NSLOT - LA) & (j < NCHUNK - LA))
            def _():
                drain_out(t)

            @pl.when(j < NCHUNK - LA)
            def _():
                issue(j + LA, t)

            drain_in(s)
            add_chunk(s)
            store(j, s)
        return carry

    lax.fori_loop(0, NCHUNK // NSLOT, body, 0)
    for s in range(NSLOT):
        drain_out(s)


@jax.jit
def _pe_add(x2d, idx1d, pe2d):
    mesh = plsc.VectorSubcoreMesh(
        core_axis_name="c", subcore_axis_name="s", num_cores=NC, num_subcores=NS
    )

    def entry(x_hbm, idx_hbm, pe_hbm, out_hbm, idx_v,
              x0, x1, x2, x3, pe0, pe1, pe2, pe3,
              si0, si1, si2, si3, so0, so1, so2, so3):
        _sc_body(x_hbm, idx_hbm, pe_hbm, out_hbm, idx_v,
                 (x0, x1, x2, x3), (pe0, pe1, pe2, pe3),
                 (si0, si1, si2, si3), (so0, so1, so2, so3))

    f = pl.kernel(
        entry,
        out_type=jax.ShapeDtypeStruct((N_ROWS, D), jnp.float32),
        mesh=mesh,
        scratch_types=[pltpu.VMEM((ROWS_PER_W,), jnp.int32)]
        + [pltpu.VMEM((C, D), jnp.float32)] * (2 * NSLOT)
        + [pltpu.SemaphoreType.DMA] * (2 * NSLOT),
    )
    return f(x2d, idx1d, pe2d)


def kernel(x, indices, pe):
    out = _pe_add(
        x.reshape(N_ROWS, D), indices.reshape(N_ROWS), pe.reshape(P, D)
    )
    return out.reshape(B, P, D)
